# baseline (speedup 1.0000x reference)
import functools

import jax
import jax.numpy as jnp
from jax import lax
from jax.experimental import pallas as pl
from jax.experimental.pallas import tpu as pltpu
from jax.experimental.pallas import tpu_sc as plsc

N_ROWS = 320000
D = 128
S = 512
NC, NS = 2, 16
NW = NC * NS
ROWS_PER_TILE = N_ROWS // NW
C = 80
NCHUNK = ROWS_PER_TILE // C
NV = 41
NST = NCHUNK - NV
NITER = NV
SEG_PER_TILE = S // NS
CW = 16
LANES = 16
NGROUP = C // LANES


def _sc_partial_segsum(x, idxp, ones_cw, ident):
  mesh = plsc.VectorSubcoreMesh(
      core_axis_name="c", subcore_axis_name="s", num_cores=NC, num_subcores=NS)

  @functools.partial(
      pl.kernel,
      out_type=(
          jax.ShapeDtypeStruct((NC * S, D), jnp.float32),
          jax.ShapeDtypeStruct((NC * S, CW), jnp.float32),
      ),
      mesh=mesh,
      compiler_params=pltpu.CompilerParams(use_tc_tiling_on_sc=False,
                                           needs_layout_passes=False),
      scratch_types=[
          pltpu.VMEM((NCHUNK, C), jnp.int32),
          [pltpu.VMEM((C, D), jnp.float32)] * 2,
          pltpu.VMEM((C, D), jnp.float32),
          pltpu.VMEM((S, D), jnp.float32),
          pltpu.VMEM((C, CW), jnp.float32),
          pltpu.VMEM((SEG_PER_TILE, CW), jnp.float32),
          pltpu.VMEM((S // 128, 128), jnp.int32),
          pltpu.VMEM_SHARED((S, D), jnp.float32),
          pltpu.VMEM_SHARED((S, CW), jnp.float32),
          [pltpu.SemaphoreType.DMA] * 2,
          pltpu.SemaphoreType.DMA,
          [pltpu.SemaphoreType.DMA] * 2,
          [pltpu.SemaphoreType.DMA] * 3,
          pltpu.SemaphoreType.DMA,
      ],
  )
  def k(x_hbm, idxp_hbm, ones_hbm, ident_hbm, psums_hbm,
        pcnts_hbm, idx_p, sbufs, vbuf, acc, ones_v, zc, ident_v,
        sums_sh, cnts_sh, gsems, vgsem, ssems, csems, msem):
    cid = lax.axis_index("c")
    sid = lax.axis_index("s")
    wid = cid * NS + sid
    row0 = wid * ROWS_PER_TILE

    pltpu.sync_copy(idxp_hbm.at[wid], idx_p)
    pltpu.sync_copy(ones_hbm.at[pl.ds(0, C)], ones_v)
    pltpu.sync_copy(ones_hbm.at[pl.ds(C, SEG_PER_TILE)], zc)
    pltpu.sync_copy(ident_hbm, ident_v)

    zeros16 = jnp.zeros((LANES,), jnp.float32)
    lane_iota = lax.iota(jnp.int32, LANES)

    def zacc(i, _):
      acc[i // (D // LANES),
          pl.ds((i % (D // LANES)) * LANES, LANES)] = zeros16
      return 0
    lax.fori_loop(0, S * (D // LANES), zacc, 0)

    pltpu.sync_copy(acc.at[pl.ds(0, SEG_PER_TILE)],
                    sums_sh.at[pl.ds(sid * SEG_PER_TILE, SEG_PER_TILE)])
    pltpu.sync_copy(zc, cnts_sh.at[pl.ds(sid * SEG_PER_TILE, SEG_PER_TILE)])
    plsc.subcore_barrier()

    def gstart(j, buf, sem):
      pltpu.async_copy(x_hbm.at[pl.ds(row0 + j * C, C)], buf, sem)

    def gwait(buf, sem):
      pltpu.make_async_copy(x_hbm.at[pl.ds(0, C)], buf, sem).wait()

    def cstart(j, ci):
      return pltpu.async_copy(ones_v, cnts_sh.at[idx_p.at[j]], csems[ci],
                              add=True)

    def reduce_chunk(j):
      def do_rows(r0, iv, nrows):
        for u in range(nrows):
          segv = jnp.full((LANES,), iv[u], jnp.int32)
          for c in range(D // LANES):
            v = vbuf[r0 + u, pl.ds(c * LANES, LANES)]
            plsc.addupdate_scatter(acc, [segv, lane_iota + (c * LANES)], v)

      def row_group(g, _):
        r0 = g * LANES
        iv = idx_p[j, pl.ds(r0, LANES)]
        do_rows(r0, iv, LANES)
        return 0
      lax.fori_loop(0, NGROUP, row_group, 0)

    gstart(0, sbufs[0], gsems[0])
    gstart(1, sbufs[1], gsems[1])
    gstart(NST, vbuf, vgsem)

    def body(i, _):
      j0 = 2 * i
      jv = NST + i
      gwait(sbufs[0], gsems[0])
      gwait(sbufs[1], gsems[1])
      d0 = pltpu.async_copy(sbufs[0], sums_sh.at[idx_p.at[j0]], ssems[0],
                            add=True)
      d1 = pltpu.async_copy(sbufs[1], sums_sh.at[idx_p.at[j0 + 1]], ssems[1],
                            add=True)
      c0 = cstart(j0, 0)
      c1 = cstart(j0 + 1, 1)
      c2 = cstart(jv, 2)
      gwait(vbuf, vgsem)
      reduce_chunk(jv)
      @pl.when(i < NITER - 1)
      def _():
        gstart(jv + 1, vbuf, vgsem)
      d0.wait()
      d1.wait()
      c0.wait()
      c1.wait()
      c2.wait()
      @pl.when(j0 + 3 < NST)
      def _():
        gstart(j0 + 2, sbufs[0], gsems[0])
        gstart(j0 + 3, sbufs[1], gsems[1])
      return 0
    lax.fori_loop(0, NITER, body, 0)

    gwait(sbufs[0], gsems[0])
    gwait(sbufs[1], gsems[1])
    dt0 = pltpu.async_copy(sbufs[0], sums_sh.at[idx_p.at[NST - 2]], ssems[0],
                           add=True)
    dt1 = pltpu.async_copy(sbufs[1], sums_sh.at[idx_p.at[NST - 1]], ssems[1],
                           add=True)
    ct0 = cstart(NST - 2, 0)
    ct1 = cstart(NST - 1, 1)
    dt0.wait()
    dt1.wait()
    ct0.wait()
    ct1.wait()

    for q in range(S // 128):
      pltpu.async_copy(acc.at[pl.ds(q * 128, 128)],
                       sums_sh.at[ident_v.at[q]],
                       msem, add=True).wait()
    plsc.subcore_barrier()

    pltpu.sync_copy(sums_sh.at[pl.ds(sid * SEG_PER_TILE, SEG_PER_TILE)],
                    acc.at[pl.ds(0, SEG_PER_TILE)])
    pltpu.sync_copy(acc.at[pl.ds(0, SEG_PER_TILE)],
                    psums_hbm.at[pl.ds(cid * S + sid * SEG_PER_TILE,
                                       SEG_PER_TILE)])
    pltpu.sync_copy(cnts_sh.at[pl.ds(sid * SEG_PER_TILE, SEG_PER_TILE)], zc)
    pltpu.sync_copy(zc, pcnts_hbm.at[pl.ds(cid * S + sid * SEG_PER_TILE,
                                           SEG_PER_TILE)])

  return k(x, idxp, ones_cw, ident)


def _combine(psums, pcnts):
  def body(ps_ref, pc_ref, o_ref):
    sums = ps_ref[0] + ps_ref[1]
    cnts = pc_ref[0, :, 0:1] + pc_ref[1, :, 0:1]
    o_ref[...] = sums / cnts
  return pl.pallas_call(
      body,
      out_shape=jax.ShapeDtypeStruct((S, D), jnp.float32),
  )(psums.reshape(NC, S, D), pcnts.reshape(NC, S, CW))


def kernel(x, batch_index):
  idxp = batch_index.astype(jnp.int32).reshape(NW, NCHUNK, C)
  ones_cw = jnp.concatenate([jnp.ones((C, CW), jnp.float32),
                             jnp.zeros((SEG_PER_TILE, CW), jnp.float32)])
  ident = jnp.arange(S, dtype=jnp.int32).reshape(S // 128, 128)
  psums, pcnts = _sc_partial_segsum(x, idxp, ones_cw, ident)
  return _combine(psums, pcnts)


# speedup vs baseline: 1.1794x; 1.1794x over previous
import functools

import jax
import jax.numpy as jnp
from jax import lax
from jax.experimental import pallas as pl
from jax.experimental.pallas import tpu as pltpu
from jax.experimental.pallas import tpu_sc as plsc

N_ROWS = 320000
D = 128
S = 512
NC, NS = 2, 16
NW = NC * NS
ROWS_PER_TILE = N_ROWS // NW
C = 125
NCHUNK = ROWS_PER_TILE // C
NBUF = 4
NGRP = NCHUNK // NBUF
SEG_PER_TILE = S // NS
CW = 16
LANES = 16


def _sc_partial_segsum(x, idx2d, ones_cw):
  mesh = plsc.VectorSubcoreMesh(
      core_axis_name="c", subcore_axis_name="s", num_cores=NC, num_subcores=NS)

  @functools.partial(
      pl.kernel,
      out_type=(
          jax.ShapeDtypeStruct((NC * S, D), jnp.float32),
          jax.ShapeDtypeStruct((NC * S, CW), jnp.float32),
      ),
      mesh=mesh,
      compiler_params=pltpu.CompilerParams(use_tc_tiling_on_sc=False),
      scratch_types=[
          pltpu.VMEM((NCHUNK, C), jnp.int32),
          [pltpu.VMEM((C, D), jnp.float32)] * NBUF,
          pltpu.VMEM((C, CW), jnp.float32),
          pltpu.VMEM((SEG_PER_TILE, CW), jnp.float32),
          pltpu.VMEM_SHARED((S, D), jnp.float32),
          pltpu.VMEM_SHARED((S, CW), jnp.float32),
          [pltpu.SemaphoreType.DMA] * NBUF,
          [pltpu.SemaphoreType.DMA] * NBUF,
          [pltpu.SemaphoreType.DMA] * NBUF,
      ],
  )
  def k(x_hbm, idx_hbm, ones_hbm, psums_hbm, pcnts_hbm, idx_v, xbufs,
        ones_v, zc, sums_sh, cnts_sh, gsems, ssems, csems):
    xbuf0 = xbufs[0]
    cid = lax.axis_index("c")
    sid = lax.axis_index("s")
    wid = cid * NS + sid
    row0 = wid * ROWS_PER_TILE

    pltpu.sync_copy(idx_hbm.at[wid], idx_v)

    zeros16 = jnp.zeros((LANES,), jnp.float32)

    def zrow(i, _):
      xbuf0[i // (D // LANES), pl.ds((i % (D // LANES)) * LANES, LANES)] = zeros16
      return 0
    lax.fori_loop(0, SEG_PER_TILE * (D // LANES), zrow, 0)
    pltpu.sync_copy(ones_hbm.at[pl.ds(0, C)], ones_v)
    pltpu.sync_copy(ones_hbm.at[pl.ds(C, SEG_PER_TILE)], zc)

    pltpu.sync_copy(xbuf0.at[pl.ds(0, SEG_PER_TILE)],
                    sums_sh.at[pl.ds(sid * SEG_PER_TILE, SEG_PER_TILE)])
    pltpu.sync_copy(zc, cnts_sh.at[pl.ds(sid * SEG_PER_TILE, SEG_PER_TILE)])
    plsc.subcore_barrier()

    def gstart(j, b):
      pltpu.async_copy(x_hbm.at[pl.ds(row0 + j * C, C)], xbufs[b], gsems[b])

    def gwait(b):
      pltpu.make_async_copy(x_hbm.at[pl.ds(0, C)], xbufs[b], gsems[b]).wait()

    for b in range(NBUF):
      gstart(b, b)

    def group(p, _):
      j0 = p * NBUF
      descs = []
      for b in range(NBUF):
        gwait(b)
        d1 = pltpu.async_copy(xbufs[b], sums_sh.at[idx_v.at[j0 + b]],
                              ssems[b], add=True)
        d2 = pltpu.async_copy(ones_v, cnts_sh.at[idx_v.at[j0 + b]],
                              csems[b], add=True)
        descs.append((d1, d2))
      for b in range(NBUF):
        d1, d2 = descs[b]
        d1.wait()
        d2.wait()
        @pl.when(p < NGRP - 1)
        def _():
          gstart(j0 + NBUF + b, b)
      return 0
    lax.fori_loop(0, NGRP, group, 0)
    plsc.subcore_barrier()

    pltpu.sync_copy(sums_sh.at[pl.ds(sid * SEG_PER_TILE, SEG_PER_TILE)],
                    xbuf0.at[pl.ds(0, SEG_PER_TILE)])
    pltpu.sync_copy(xbuf0.at[pl.ds(0, SEG_PER_TILE)],
                    psums_hbm.at[pl.ds(cid * S + sid * SEG_PER_TILE,
                                       SEG_PER_TILE)])
    pltpu.sync_copy(cnts_sh.at[pl.ds(sid * SEG_PER_TILE, SEG_PER_TILE)], zc)
    pltpu.sync_copy(zc, pcnts_hbm.at[pl.ds(cid * S + sid * SEG_PER_TILE,
                                           SEG_PER_TILE)])

  return k(x, idx2d, ones_cw)


def _combine(psums, pcnts):
  def body(ps_ref, pc_ref, o_ref):
    sums = ps_ref[0] + ps_ref[1]
    cnts = pc_ref[0, :, 0:1] + pc_ref[1, :, 0:1]
    o_ref[...] = sums / cnts
  return pl.pallas_call(
      body,
      out_shape=jax.ShapeDtypeStruct((S, D), jnp.float32),
  )(psums.reshape(NC, S, D), pcnts.reshape(NC, S, CW))


def kernel(x, batch_index):
  idx2d = batch_index.astype(jnp.int32).reshape(NW, NCHUNK, C)
  ones_cw = jnp.concatenate([jnp.ones((C, CW), jnp.float32),
                             jnp.zeros((SEG_PER_TILE, CW), jnp.float32)])
  psums, pcnts = _sc_partial_segsum(x, idx2d, ones_cw)
  return _combine(psums, pcnts)
